# baseline (speedup 1.0000x reference)
import functools

import jax
import jax.numpy as jnp
from jax import lax
from jax.experimental import pallas as pl
from jax.experimental.pallas import tpu as pltpu
from jax.experimental.pallas import tpu_sc as plsc

_NUM_WORKERS = 32
_CHUNK_B = 8
_NBUF = 2
_NIDX = 3


def _gather_kernel(batch, seq, emb):
  rows_per_step = _CHUNK_B * seq
  per_w_b = batch // _NUM_WORKERS
  n_chunks = per_w_b // _CHUNK_B
  mesh = plsc.VectorSubcoreMesh(core_axis_name="c", subcore_axis_name="s")

  @functools.partial(
      pl.kernel,
      mesh=mesh,
      out_type=jax.ShapeDtypeStruct((batch, seq, emb), jnp.float32),
      scratch_types=[
          pltpu.VMEM((_NIDX, rows_per_step), jnp.int32),
          pltpu.VMEM((_NBUF, _CHUNK_B * seq, emb), jnp.float32),
          [pltpu.SemaphoreType.DMA] * _NIDX,
          [pltpu.SemaphoreType.DMA] * _NBUF,
          [pltpu.SemaphoreType.DMA] * _NBUF,
      ],
      compiler_params=pltpu.CompilerParams(use_tc_tiling_on_sc=False),
  )
  def k(idx_hbm, table_hbm, out_hbm, idx_v, rows_v, si, sg, sw):
    wid = lax.axis_index("s") * 2 + lax.axis_index("c")
    base_r = wid * per_w_b * seq
    base_b = wid * per_w_b

    def fire_idx(i):
      b = i % _NIDX
      return pltpu.async_copy(
          idx_hbm.at[pl.ds(base_r + i * rows_per_step, rows_per_step)],
          idx_v.at[b], si[b])

    def fire_gather(i, b):
      return pltpu.async_copy(table_hbm.at[idx_v.at[i % _NIDX]], rows_v.at[b],
                              sg[b])

    idx_h = [None] * _NIDX
    g = [None] * _NBUF
    w = [None] * _NBUF

    idx_h[0] = fire_idx(0)
    if n_chunks > 1:
      idx_h[1] = fire_idx(1)
    idx_h[0].wait()
    g[0] = fire_gather(0, 0)

    for i in range(n_chunks):
      cur = i % _NBUF
      nxt = (i + 1) % _NBUF
      if i + 2 < n_chunks:
        idx_h[(i + 2) % _NIDX] = fire_idx(i + 2)
      if i + 1 < n_chunks:
        idx_h[(i + 1) % _NIDX].wait()
        if w[nxt] is not None:
          for h in w[nxt]:
            h.wait()
          w[nxt] = None
        g[nxt] = fire_gather(i + 1, nxt)
      g[cur].wait()
      w[cur] = [
          pltpu.async_copy(rows_v.at[cur, pl.ds(u * seq, seq)],
                           out_hbm.at[base_b + i * _CHUNK_B + u], sw[cur])
          for u in range(_CHUNK_B)
      ]

    for b in range(_NBUF):
      if w[b] is not None:
        for h in w[b]:
          h.wait()

  return k


def kernel(tokens, table):
  batch, seq = tokens.shape
  vocab, emb = table.shape
  flat = tokens.reshape(batch * seq).astype(jnp.int32)
  return _gather_kernel(batch, seq, emb)(flat, table)


# speedup vs baseline: 1.0558x; 1.0558x over previous
import functools

import jax
import jax.numpy as jnp
from jax import lax
from jax.experimental import pallas as pl
from jax.experimental.pallas import tpu as pltpu
from jax.experimental.pallas import tpu_sc as plsc

_NUM_WORKERS = 32
_SCHUNK = 8
_NBUF = 2
_NIDX = 3


def _gather_kernel(batch, seq, emb):
  bblk = batch // _NUM_WORKERS
  n_chunks = seq // _SCHUNK
  mesh = plsc.VectorSubcoreMesh(core_axis_name="c", subcore_axis_name="s")

  @functools.partial(
      pl.kernel,
      mesh=mesh,
      out_type=jax.ShapeDtypeStruct((seq, batch, emb), jnp.float32),
      scratch_types=[
          pltpu.VMEM((_NIDX, _SCHUNK, bblk), jnp.int32),
          pltpu.VMEM((_NBUF, _SCHUNK, bblk, emb), jnp.float32),
          [pltpu.SemaphoreType.DMA] * _NIDX,
          [pltpu.SemaphoreType.DMA] * _NBUF,
          [pltpu.SemaphoreType.DMA] * _NBUF,
      ],
      compiler_params=pltpu.CompilerParams(use_tc_tiling_on_sc=False),
  )
  def k(tok_hbm, table_hbm, out_hbm, idx_v, rows_v, si, sg, sw):
    wid = lax.axis_index("s") * 2 + lax.axis_index("c")
    b0 = wid * bblk

    def fire_idx(i):
      b = i % _NIDX
      return pltpu.async_copy(
          tok_hbm.at[pl.ds(i * _SCHUNK, _SCHUNK), pl.ds(b0, bblk)],
          idx_v.at[b], si[b])

    def fire_gathers(i, b):
      return [
          pltpu.async_copy(table_hbm.at[idx_v.at[i % _NIDX, u]],
                           rows_v.at[b, u], sg[b])
          for u in range(_SCHUNK)
      ]

    def fire_writes(i, b):
      return [
          pltpu.async_copy(rows_v.at[b, u],
                           out_hbm.at[i * _SCHUNK + u, pl.ds(b0, bblk)],
                           sw[b])
          for u in range(_SCHUNK)
      ]

    idx_h = [None] * _NIDX
    g = [None] * _NBUF
    w = [None] * _NBUF

    idx_h[0] = fire_idx(0)
    if n_chunks > 1:
      idx_h[1] = fire_idx(1)
    idx_h[0].wait()
    g[0] = fire_gathers(0, 0)

    for i in range(n_chunks):
      cur = i % _NBUF
      nxt = (i + 1) % _NBUF
      if i + 2 < n_chunks:
        idx_h[(i + 2) % _NIDX] = fire_idx(i + 2)
      if i + 1 < n_chunks:
        idx_h[(i + 1) % _NIDX].wait()
        if w[nxt] is not None:
          for h in w[nxt]:
            h.wait()
          w[nxt] = None
        g[nxt] = fire_gathers(i + 1, nxt)
      for h in g[cur]:
        h.wait()
      w[cur] = fire_writes(i, cur)

    for b in range(_NBUF):
      if w[b] is not None:
        for h in w[b]:
          h.wait()

  return k


def kernel(tokens, table):
  batch, seq = tokens.shape
  vocab, emb = table.shape
  tok_t = tokens.T.astype(jnp.int32)
  out_t = _gather_kernel(batch, seq, emb)(tok_t, table)
  return jnp.transpose(out_t, (1, 0, 2))
